# Initial kernel scaffold; baseline (speedup 1.0000x reference)
import jax
import jax.numpy as jnp
from jax.experimental import pallas as pl


def kernel(xyz):
    raise NotImplementedError("write your pallas kernel here")


# speedup vs baseline: 2.0690x; 2.0690x over previous
import functools

import jax
import jax.numpy as jnp
from jax.experimental import pallas as pl

B, N, D = 8, 8192, 24
G = 512
K = 32
NR, NC = 64, 128
GT = 64
BIG = 1e10


def _tree24(q):
    s = [(q[k] + q[k + 8]) + q[k + 16] for k in range(8)]
    return ((s[0] + s[4]) + (s[2] + s[6])) + ((s[1] + s[5]) + (s[3] + s[7]))


def _fps_kernel(rows_ref, t3_ref, centers_ref):
    iota = (jax.lax.broadcasted_iota(jnp.int32, (NR, NC), 0) * NC
            + jax.lax.broadcasted_iota(jnp.int32, (NR, NC), 1))

    def body(i, carry):
        fs, dists = carry
        new_fs, new_dists = [], []
        for b in range(B):
            c_row = rows_ref[b, pl.ds(fs[b], 1), :]
            centers_ref[b, pl.ds(i, 1), :] = c_row
            cc = jnp.reshape(jnp.swapaxes(c_row, 0, 1), (D, 1, 1))
            diff = t3_ref[b] - cc
            q = diff * diff
            dist = _tree24([q[d] for d in range(D)])
            db = jnp.minimum(dists[b], dist)
            m = jnp.max(db)
            new_fs.append(jnp.min(jnp.where(db == m, iota, N)))
            new_dists.append(db)
        return tuple(new_fs), tuple(new_dists)

    f0 = (jnp.int32(0),) * B
    dists0 = (jnp.full((NR, NC), BIG, jnp.float32),) * B
    jax.lax.fori_loop(0, G, body, (f0, dists0))


def _knn_kernel(rows_ref, lanes_ref, centers_ref, out_ref):
    rows = rows_ref[0]
    c = centers_ref[0]

    x2 = _tree24([lanes_ref[0, d:d + 1, :] * lanes_ref[0, d:d + 1, :]
                  for d in range(D)])
    q2 = _tree24([c[:, d:d + 1] * c[:, d:d + 1]
                  for d in range(D)])
    e = jax.lax.dot_general(c, rows, (((1,), (1,)), ((), ())),
                            preferred_element_type=jnp.float32)
    d2 = (q2 + x2) - 2.0 * e

    iota_n = jax.lax.broadcasted_iota(jnp.int32, (GT, N), 1)

    def body(j, d2):
        m = jnp.min(d2, axis=1, keepdims=True)
        loc = jnp.min(jnp.where(d2 == m, iota_n, N), axis=1,
                      keepdims=True)
        onehot = iota_n == loc
        sel = onehot.astype(jnp.float32)
        gathered = jax.lax.dot_general(
            sel, rows, (((1,), (0,)), ((), ())),
            precision=jax.lax.Precision.HIGHEST,
            preferred_element_type=jnp.float32)
        out_ref[0, :, pl.ds(j, 1), :] = (gathered - c)[:, None, :]
        return jnp.where(onehot, jnp.inf, d2)

    jax.lax.fori_loop(0, K, body, d2)


@jax.jit
def kernel(xyz):
    xyz = jnp.asarray(xyz, jnp.float32)
    xyz_T = jnp.swapaxes(xyz, 1, 2)
    xyz_t3 = xyz_T.reshape(B, D, NR, NC)

    centers = pl.pallas_call(
        _fps_kernel,
        grid=(1,),
        in_specs=[
            pl.BlockSpec((B, N, D), lambda _: (0, 0, 0)),
            pl.BlockSpec((B, D, NR, NC), lambda _: (0, 0, 0, 0)),
        ],
        out_specs=pl.BlockSpec((B, G, D), lambda _: (0, 0, 0)),
        out_shape=jax.ShapeDtypeStruct((B, G, D), jnp.float32),
    )(xyz, xyz_t3)

    neighborhood = pl.pallas_call(
        _knn_kernel,
        grid=(B, G // GT),
        in_specs=[
            pl.BlockSpec((1, N, D), lambda b, t: (b, 0, 0)),
            pl.BlockSpec((1, D, N), lambda b, t: (b, 0, 0)),
            pl.BlockSpec((1, GT, D), lambda b, t: (b, t, 0)),
        ],
        out_specs=pl.BlockSpec((1, GT, K, D), lambda b, t: (b, t, 0, 0)),
        out_shape=jax.ShapeDtypeStruct((B, G, K, D), jnp.float32),
    )(xyz, xyz_T, centers)

    return neighborhood, centers
